# baseline (speedup 1.0000x reference)
import functools

import jax
import jax.numpy as jnp
from jax.experimental import pallas as pl
from jax.experimental.pallas import tpu as pltpu


def _se_kernel(x_ref, w1_ref, b1_ref, w2_ref, b2_ref, o_ref, *, inv_hw):
    x = x_ref[...]
    pooled = jnp.sum(x, axis=-1) * inv_hw
    h = jax.lax.dot_general(
        pooled, w1_ref[...], (((1,), (1,)), ((), ())),
        preferred_element_type=jnp.float32) + b1_ref[...]
    h = jnp.maximum(h, 0.0)
    s = jax.lax.dot_general(
        h, w2_ref[...], (((1,), (1,)), ((), ())),
        preferred_element_type=jnp.float32) + b2_ref[...]
    gate = jax.nn.sigmoid(s)
    o_ref[...] = jnp.broadcast_to(gate[:, :, None], o_ref.shape)


def kernel(x, w1, b1, w2, b2):
    B, C, H, W = x.shape
    hidden = w1.shape[0]
    HW = H * W

    bb = 2
    while B % bb:
        bb //= 2

    x3 = x.reshape(B, C, HW)
    out = pl.pallas_call(
        functools.partial(_se_kernel, inv_hw=1.0 / HW),
        out_shape=jax.ShapeDtypeStruct((B, C, HW), x.dtype),
        grid=(B // bb,),
        in_specs=[
            pl.BlockSpec((bb, C, HW), lambda i: (i, 0, 0)),
            pl.BlockSpec((hidden, C), lambda i: (0, 0)),
            pl.BlockSpec((1, hidden), lambda i: (0, 0)),
            pl.BlockSpec((C, hidden), lambda i: (0, 0)),
            pl.BlockSpec((1, C), lambda i: (0, 0)),
        ],
        out_specs=pl.BlockSpec((bb, C, HW), lambda i: (i, 0, 0)),
        compiler_params=pltpu.CompilerParams(
            dimension_semantics=("parallel",),
            vmem_limit_bytes=64 * 1024 * 1024,
        ),
    )(x3, w1, b1.reshape(1, hidden), w2, b2.reshape(1, C))

    return out.reshape(B, C, H, W)


# speedup vs baseline: 1.0640x; 1.0640x over previous
import functools

import jax
import jax.numpy as jnp
from jax.experimental import pallas as pl
from jax.experimental.pallas import tpu as pltpu


def _se_kernel(x_ref, w1_ref, b1_ref, w2_ref, b2_ref, o_ref, *, inv_hw):
    x = x_ref[...]
    pooled = jnp.sum(x, axis=-1) * inv_hw
    h = jax.lax.dot_general(
        pooled, w1_ref[...], (((1,), (1,)), ((), ())),
        preferred_element_type=jnp.float32) + b1_ref[...]
    h = jnp.maximum(h, 0.0)
    s = jax.lax.dot_general(
        h, w2_ref[...], (((1,), (1,)), ((), ())),
        preferred_element_type=jnp.float32) + b2_ref[...]
    gate = jax.nn.sigmoid(s)
    o_ref[...] = jnp.broadcast_to(gate[:, :, None], o_ref.shape)


def kernel(x, w1, b1, w2, b2):
    B, C, H, W = x.shape
    hidden = w1.shape[0]
    HW = H * W

    bb = 8
    while B % bb:
        bb //= 2

    x3 = x.reshape(B, C, HW)
    out = pl.pallas_call(
        functools.partial(_se_kernel, inv_hw=1.0 / HW),
        out_shape=jax.ShapeDtypeStruct((B, C, HW), x.dtype),
        grid=(B // bb,),
        in_specs=[
            pl.BlockSpec((bb, C, HW), lambda i: (i, 0, 0)),
            pl.BlockSpec((hidden, C), lambda i: (0, 0)),
            pl.BlockSpec((1, hidden), lambda i: (0, 0)),
            pl.BlockSpec((C, hidden), lambda i: (0, 0)),
            pl.BlockSpec((1, C), lambda i: (0, 0)),
        ],
        out_specs=pl.BlockSpec((bb, C, HW), lambda i: (i, 0, 0)),
        compiler_params=pltpu.CompilerParams(
            dimension_semantics=("parallel",),
            vmem_limit_bytes=64 * 1024 * 1024,
        ),
    )(x3, w1, b1.reshape(1, hidden), w2, b2.reshape(1, C))

    return out.reshape(B, C, H, W)
